# baseline (speedup 1.0000x reference)
import jax
import jax.numpy as jnp
from jax.experimental import pallas as pl
from jax.experimental.pallas import tpu as pltpu

K = 1024
D = 64
NC = 8
NT = 2


def _vq_body(c_ref, z_ref, cb_ref, zq_ref, qb_ref):
    z = z_ref[0]
    cb = cb_ref[0]
    a = jnp.sum(z * z, axis=0, keepdims=True)
    e = jax.lax.dot_general(cb, z, (((1,), (0,)), ((), ())),
                            precision=jax.lax.Precision.DEFAULT)
    b2 = jnp.sum(cb * cb, axis=-1, keepdims=True)
    dists = a - 2.0 * e + b2
    m = jnp.min(dists, axis=0, keepdims=True)
    iota = jax.lax.broadcasted_iota(jnp.int32, dists.shape, 0)
    masked = jnp.where(dists == m, iota, K)
    idx = jnp.min(masked, axis=0, keepdims=True)
    onehot = (masked == idx).astype(jnp.bfloat16)
    cb_hi = cb.astype(jnp.bfloat16)
    cb_lo = (cb - cb_hi.astype(jnp.float32)).astype(jnp.bfloat16)
    chl = jnp.concatenate([cb_hi, cb_lo], axis=1)
    dn = (((0,), (0,)), ((), ()))
    qhl = jax.lax.dot_general(chl, onehot, dn,
                              preferred_element_type=jnp.float32)
    quant = qhl[:D] + qhl[D:]
    zq_ref[0] = z + (quant - z)
    qb_ref[0] = quant


def kernel(z_e_x, C, emb_weight):
    B, Dd, H, W = z_e_x.shape
    HW = H * W
    HWt = HW // NT
    z = z_e_x.reshape(B, Dd, HW)
    grid_spec = pltpu.PrefetchScalarGridSpec(
        num_scalar_prefetch=1,
        grid=(B, NT),
        in_specs=[
            pl.BlockSpec((1, Dd, HWt), lambda b, t, c: (b, 0, t)),
            pl.BlockSpec((1, K, Dd), lambda b, t, c: (c[b], 0, 0)),
        ],
        out_specs=[
            pl.BlockSpec((1, Dd, HWt), lambda b, t, c: (b, 0, t)),
            pl.BlockSpec((1, Dd, HWt), lambda b, t, c: (b, 0, t)),
        ],
    )
    zq, qb = pl.pallas_call(
        _vq_body,
        grid_spec=grid_spec,
        out_shape=[
            jax.ShapeDtypeStruct((B, Dd, HW), jnp.float32),
            jax.ShapeDtypeStruct((B, Dd, HW), jnp.float32),
        ],
    )(C, z, emb_weight)
    return zq.reshape(B, Dd, H, W), qb.reshape(B, Dd, H, W)


# speedup vs baseline: 1.1061x; 1.1061x over previous
import jax
import jax.numpy as jnp
from jax.experimental import pallas as pl
from jax.experimental.pallas import tpu as pltpu

K = 1024
D = 64
NC = 8
NT = 1


def _vq_body(c_ref, z_ref, cb_ref, zq_ref, qb_ref):
    z = z_ref[0]
    cb = cb_ref[0]
    a = jnp.sum(z * z, axis=0, keepdims=True)
    e = jax.lax.dot_general(cb, z, (((1,), (0,)), ((), ())),
                            precision=jax.lax.Precision.DEFAULT)
    b2 = jnp.sum(cb * cb, axis=-1, keepdims=True)
    dists = a - 2.0 * e + b2
    m = jnp.min(dists, axis=0, keepdims=True)
    iota = jax.lax.broadcasted_iota(jnp.int32, dists.shape, 0)
    masked = jnp.where(dists == m, iota, K)
    idx = jnp.min(masked, axis=0, keepdims=True)
    onehot = (masked == idx).astype(jnp.bfloat16)
    cb_hi = cb.astype(jnp.bfloat16)
    cb_lo = (cb - cb_hi.astype(jnp.float32)).astype(jnp.bfloat16)
    chl = jnp.concatenate([cb_hi, cb_lo], axis=1)
    dn = (((0,), (0,)), ((), ()))
    qhl = jax.lax.dot_general(chl, onehot, dn,
                              preferred_element_type=jnp.float32)
    quant = qhl[:D] + qhl[D:]
    zq_ref[0] = z + (quant - z)
    qb_ref[0] = quant


def kernel(z_e_x, C, emb_weight):
    B, Dd, H, W = z_e_x.shape
    HW = H * W
    HWt = HW // NT
    z = z_e_x.reshape(B, Dd, HW)
    grid_spec = pltpu.PrefetchScalarGridSpec(
        num_scalar_prefetch=1,
        grid=(B, NT),
        in_specs=[
            pl.BlockSpec((1, Dd, HWt), lambda b, t, c: (b, 0, t)),
            pl.BlockSpec((1, K, Dd), lambda b, t, c: (c[b], 0, 0)),
        ],
        out_specs=[
            pl.BlockSpec((1, Dd, HWt), lambda b, t, c: (b, 0, t)),
            pl.BlockSpec((1, Dd, HWt), lambda b, t, c: (b, 0, t)),
        ],
    )
    zq, qb = pl.pallas_call(
        _vq_body,
        grid_spec=grid_spec,
        out_shape=[
            jax.ShapeDtypeStruct((B, Dd, HW), jnp.float32),
            jax.ShapeDtypeStruct((B, Dd, HW), jnp.float32),
        ],
    )(C, z, emb_weight)
    return zq.reshape(B, Dd, H, W), qb.reshape(B, Dd, H, W)
